# Initial kernel scaffold; baseline (speedup 1.0000x reference)
import jax
import jax.numpy as jnp
from jax.experimental import pallas as pl


def kernel(x, edge_index, batch, W1, b1, W2, b2):
    raise NotImplementedError("write your pallas kernel here")


# speedup vs baseline: 12.1489x; 12.1489x over previous
import functools

import jax
import jax.numpy as jnp
from jax import lax
from jax.experimental import pallas as pl
from jax.experimental.pallas import tpu as pltpu
from jax.experimental.pallas import tpu_sc as plsc

_N = 10000
_E = 320000
_D = 128
_G = 16
_NC = 2
_NS = 16
_NW = _NC * _NS
_K = 80
_EPW = _E // _NW
_ITERS = _EPW // _K
_NP = 10240
_RPT = _NP // _NS
_DW = _D
_R = 1000

def _deg_body(dst_hbm, ones_hbm, zeros_hbm, out_hbm, didx, ones_v, acc):
    c = lax.axis_index("c")
    s = lax.axis_index("s")
    wid = c * _NS + s
    pltpu.sync_copy(zeros_hbm.at[pl.ds(s * _RPT, _RPT)],
                    acc.at[pl.ds(s * _RPT, _RPT)])
    pltpu.sync_copy(ones_hbm, ones_v)
    plsc.subcore_barrier()

    def body(i, carry):
        base = wid * _EPW + i * _K
        pltpu.sync_copy(dst_hbm.at[pl.ds(base, _K)], didx)
        pltpu.sync_copy(ones_v, acc.at[didx], add=True)
        return carry

    lax.fori_loop(0, _ITERS, body, 0)
    plsc.subcore_barrier()
    pltpu.sync_copy(acc.at[pl.ds(s * _RPT, _RPT)],
                    out_hbm.at[c, pl.ds(s * _RPT, _RPT)])


@functools.lru_cache(maxsize=None)
def _deg_call():
    mesh = plsc.VectorSubcoreMesh(core_axis_name="c", subcore_axis_name="s")
    return pl.kernel(
        _deg_body,
        out_type=jax.ShapeDtypeStruct((_NC, _NP, _DW), jnp.float32),
        mesh=mesh,
        scratch_types=[
            pltpu.VMEM((_K,), jnp.int32),
            pltpu.VMEM((_K, _DW), jnp.float32),
            pltpu.VMEM_SHARED((_NP, _DW), jnp.float32),
        ],
    )


def _edge_body(g_hbm, src_hbm, dst_hbm, zeros_hbm, out_hbm,
               sidx, didx, rows, acc, sem):
    c = lax.axis_index("c")
    s = lax.axis_index("s")
    wid = c * _NS + s
    pltpu.sync_copy(zeros_hbm.at[pl.ds(s * _RPT, _RPT)],
                    acc.at[pl.ds(s * _RPT, _RPT)])
    plsc.subcore_barrier()

    def body(i, carry):
        base = wid * _EPW + i * _K
        pltpu.sync_copy(src_hbm.at[pl.ds(base, _K)], sidx)
        pltpu.sync_copy(dst_hbm.at[pl.ds(base, _K)], didx)
        pltpu.async_copy(g_hbm.at[sidx], rows, sem).wait()
        pltpu.sync_copy(rows, acc.at[didx], add=True)
        return carry

    lax.fori_loop(0, _ITERS, body, 0)
    plsc.subcore_barrier()
    pltpu.sync_copy(acc.at[pl.ds(s * _RPT, _RPT)],
                    out_hbm.at[c, pl.ds(s * _RPT, _RPT)])


@functools.lru_cache(maxsize=None)
def _edge_call():
    mesh = plsc.VectorSubcoreMesh(core_axis_name="c", subcore_axis_name="s")
    return pl.kernel(
        _edge_body,
        out_type=jax.ShapeDtypeStruct((_NC, _NP, _D), jnp.float32),
        mesh=mesh,
        scratch_types=[
            pltpu.VMEM((_K,), jnp.int32),
            pltpu.VMEM((_K,), jnp.int32),
            pltpu.VMEM((_K, _D), jnp.float32),
            pltpu.VMEM_SHARED((_NP, _D), jnp.float32),
            pltpu.SemaphoreType.DMA,
        ],
    )


def _mm1_body(d_ref, x_ref, w_ref, g_ref, dinv_ref):
    db = d_ref[...]
    deg = db[0, :, :1] + db[1, :, :1] + 1.0
    dinv = lax.rsqrt(deg)
    hw = jnp.dot(x_ref[...], w_ref[...], preferred_element_type=jnp.float32)
    g_ref[...] = hw * dinv
    dinv_ref[...] = dinv


def _tc1(degp, x, W1):
    return pl.pallas_call(
        _mm1_body,
        grid=(_N // _R,),
        in_specs=[
            pl.BlockSpec((_NC, _R, _DW), lambda j: (0, j, 0)),
            pl.BlockSpec((_R, _D), lambda j: (j, 0)),
            pl.BlockSpec((_D, _D), lambda j: (0, 0)),
        ],
        out_specs=[
            pl.BlockSpec((_R, _D), lambda j: (j, 0)),
            pl.BlockSpec((_R, 1), lambda j: (j, 0)),
        ],
        out_shape=[
            jax.ShapeDtypeStruct((_N, _D), jnp.float32),
            jax.ShapeDtypeStruct((_N, 1), jnp.float32),
        ],
    )(degp, x, W1)


def _mid_body(acc_ref, g_ref, dinv_ref, b_ref, w_ref, out_ref):
    a = acc_ref[...]
    ssum = a[0] + a[1] + g_ref[...]
    h = jnp.maximum(ssum * dinv_ref[...] + b_ref[...], 0.0)
    out_ref[...] = jnp.dot(h, w_ref[...],
                           preferred_element_type=jnp.float32) * dinv_ref[...]


def _tc2(acc, g, dinv, b, W2):
    return pl.pallas_call(
        _mid_body,
        grid=(_N // _R,),
        in_specs=[
            pl.BlockSpec((_NC, _R, _D), lambda j: (0, j, 0)),
            pl.BlockSpec((_R, _D), lambda j: (j, 0)),
            pl.BlockSpec((_R, 1), lambda j: (j, 0)),
            pl.BlockSpec((1, _D), lambda j: (0, 0)),
            pl.BlockSpec((_D, _D), lambda j: (0, 0)),
        ],
        out_specs=pl.BlockSpec((_R, _D), lambda j: (j, 0)),
        out_shape=jax.ShapeDtypeStruct((_N, _D), jnp.float32),
    )(acc, g, dinv, b, W2)


def _fin_body(acc_ref, g_ref, dinv_ref, b_ref, batch_ref, out_ref):
    j = pl.program_id(0)
    a = acc_ref[...]
    h = (a[0] + a[1] + g_ref[...]) * dinv_ref[...] + b_ref[...]
    bb = batch_ref[...]

    @pl.when(j == 0)
    def _():
        out_ref[...] = jnp.full((_G, _D), -jnp.inf, jnp.float32)

    for gi in range(_G):
        vals = jnp.where(bb == gi, h, -jnp.inf)
        m = jnp.max(vals, axis=0)
        out_ref[gi, :] = jnp.maximum(out_ref[gi, :], m)


def _tc3(acc, g, dinv, b, batch2d):
    return pl.pallas_call(
        _fin_body,
        grid=(_N // _R,),
        in_specs=[
            pl.BlockSpec((_NC, _R, _D), lambda j: (0, j, 0)),
            pl.BlockSpec((_R, _D), lambda j: (j, 0)),
            pl.BlockSpec((_R, 1), lambda j: (j, 0)),
            pl.BlockSpec((1, _D), lambda j: (0, 0)),
            pl.BlockSpec((_R, 1), lambda j: (j, 0)),
        ],
        out_specs=pl.BlockSpec((_G, _D), lambda j: (0, 0)),
        out_shape=jax.ShapeDtypeStruct((_G, _D), jnp.float32),
    )(acc, g, dinv, b, batch2d)


def kernel(x, edge_index, batch, W1, b1, W2, b2):
    src = edge_index[0]
    dst = edge_index[1]
    zeros_nd = jnp.zeros((_NP, _D), jnp.float32)
    ones_kw = jnp.ones((_K, _DW), jnp.float32)

    degp = _deg_call()(dst, ones_kw, zeros_nd)
    g1, dinv = _tc1(degp, x, W1)
    acc1 = _edge_call()(g1, src, dst, zeros_nd)
    g2 = _tc2(acc1, g1, dinv, b1.reshape(1, _D), W2)
    acc2 = _edge_call()(g2, src, dst, zeros_nd)
    out = _tc3(acc2, g2, dinv, b2.reshape(1, _D), batch.reshape(_N, 1))
    return out


# speedup vs baseline: 18.9095x; 1.5565x over previous
import functools

import jax
import jax.numpy as jnp
from jax import lax
from jax.experimental import pallas as pl
from jax.experimental.pallas import tpu as pltpu
from jax.experimental.pallas import tpu_sc as plsc

_N = 10000
_E = 320000
_D = 128
_G = 16
_NC = 2
_NS = 16
_NW = _NC * _NS
_K = 32
_EP = 327680
_CHT = 320
_SB = 4
_NIT = _CHT // (2 * _SB)
_NP = 10240
_RPT = _NP // _NS
_R = 1000


def _deg_body(eidx_hbm, ones_hbm, zeros_hbm, out_hbm, ib0, ib1, ones_v, acc, ssem):
    c = lax.axis_index("c")
    s = lax.axis_index("s")
    w = c * _NS + s
    pltpu.sync_copy(zeros_hbm.at[pl.ds(s * _RPT, _RPT)],
                    acc.at[pl.ds(s * _RPT, _RPT)])
    pltpu.sync_copy(ones_hbm, ones_v)
    plsc.subcore_barrier()

    def body(j, carry):
        for p, ib in ((0, ib0), (1, ib1)):
            @pl.when(j != 0)
            def _():
                for t in range(_SB):
                    pltpu.make_async_copy(
                        ones_v, acc.at[ib.at[2 * t + 1]], ssem).wait()
            pltpu.sync_copy(
                eidx_hbm.at[w, pl.ds(16 * j + 8 * p, 2 * _SB)], ib)
            for t in range(_SB):
                pltpu.async_copy(
                    ones_v, acc.at[ib.at[2 * t + 1]], ssem, add=True)
        return carry

    lax.fori_loop(0, _NIT, body, 0)
    for ib in (ib0, ib1):
        for t in range(_SB):
            pltpu.make_async_copy(ones_v, acc.at[ib.at[2 * t + 1]], ssem).wait()
    plsc.subcore_barrier()
    pltpu.sync_copy(acc.at[pl.ds(s * _RPT, _RPT)],
                    out_hbm.at[c, pl.ds(s * _RPT, _RPT)])


@functools.lru_cache(maxsize=None)
def _deg_call():
    mesh = plsc.VectorSubcoreMesh(core_axis_name="c", subcore_axis_name="s")
    return pl.kernel(
        _deg_body,
        out_type=jax.ShapeDtypeStruct((_NC, _NP, _D), jnp.float32),
        mesh=mesh,
        scratch_types=[
            pltpu.VMEM((2 * _SB, _K), jnp.int32),
            pltpu.VMEM((2 * _SB, _K), jnp.int32),
            pltpu.VMEM((_K, _D), jnp.float32),
            pltpu.VMEM_SHARED((_NP, _D), jnp.float32),
            pltpu.SemaphoreType.DMA,
        ],
    )


def _edge_body(g_hbm, eidx_hbm, zeros_hbm, out_hbm,
               ib0, ib1, r0, r1, r2, r3, r4, r5, r6, r7, acc, gsem, ssem):
    c = lax.axis_index("c")
    s = lax.axis_index("s")
    w = c * _NS + s
    rows = (r0, r1, r2, r3, r4, r5, r6, r7)
    pltpu.sync_copy(zeros_hbm.at[pl.ds(s * _RPT, _RPT)],
                    acc.at[pl.ds(s * _RPT, _RPT)])
    plsc.subcore_barrier()

    def body(j, carry):
        for p, ib in ((0, ib0), (1, ib1)):
            bufs = rows[_SB * p:_SB * (p + 1)]

            @pl.when(j != 0)
            def _():
                for t in range(_SB):
                    pltpu.make_async_copy(
                        bufs[t], acc.at[ib.at[2 * t + 1]], ssem).wait()
            pltpu.sync_copy(
                eidx_hbm.at[w, pl.ds(16 * j + 8 * p, 2 * _SB)], ib)
            for t in range(_SB):
                pltpu.async_copy(g_hbm.at[ib.at[2 * t]], bufs[t], gsem)
            for t in range(_SB):
                pltpu.make_async_copy(
                    g_hbm.at[ib.at[2 * t]], bufs[t], gsem).wait()
            for t in range(_SB):
                pltpu.async_copy(
                    bufs[t], acc.at[ib.at[2 * t + 1]], ssem, add=True)
        return carry

    lax.fori_loop(0, _NIT, body, 0)
    for p, ib in ((0, ib0), (1, ib1)):
        for t in range(_SB):
            pltpu.make_async_copy(
                rows[_SB * p + t], acc.at[ib.at[2 * t + 1]], ssem).wait()
    plsc.subcore_barrier()
    pltpu.sync_copy(acc.at[pl.ds(s * _RPT, _RPT)],
                    out_hbm.at[c, pl.ds(s * _RPT, _RPT)])


@functools.lru_cache(maxsize=None)
def _edge_call():
    mesh = plsc.VectorSubcoreMesh(core_axis_name="c", subcore_axis_name="s")
    return pl.kernel(
        _edge_body,
        out_type=jax.ShapeDtypeStruct((_NC, _NP, _D), jnp.float32),
        mesh=mesh,
        scratch_types=[
            pltpu.VMEM((2 * _SB, _K), jnp.int32),
            pltpu.VMEM((2 * _SB, _K), jnp.int32),
        ] + [pltpu.VMEM((_K, _D), jnp.float32)] * 8 + [
            pltpu.VMEM_SHARED((_NP, _D), jnp.float32),
            pltpu.SemaphoreType.DMA,
            pltpu.SemaphoreType.DMA,
        ],
    )


def _mm1_body(d_ref, x_ref, w_ref, g_ref, dinv_ref):
    db = d_ref[...]
    deg = db[0, :, :1] + db[1, :, :1] + 1.0
    dinv = lax.rsqrt(deg)
    hw = jnp.dot(x_ref[...], w_ref[...], preferred_element_type=jnp.float32)
    g_ref[...] = hw * dinv
    dinv_ref[...] = dinv


def _tc1(degp, x, W1):
    return pl.pallas_call(
        _mm1_body,
        grid=(_N // _R,),
        in_specs=[
            pl.BlockSpec((_NC, _R, _D), lambda j: (0, j, 0)),
            pl.BlockSpec((_R, _D), lambda j: (j, 0)),
            pl.BlockSpec((_D, _D), lambda j: (0, 0)),
        ],
        out_specs=[
            pl.BlockSpec((_R, _D), lambda j: (j, 0)),
            pl.BlockSpec((_R, 1), lambda j: (j, 0)),
        ],
        out_shape=[
            jax.ShapeDtypeStruct((_N, _D), jnp.float32),
            jax.ShapeDtypeStruct((_N, 1), jnp.float32),
        ],
    )(degp, x, W1)


def _mid_body(acc_ref, g_ref, dinv_ref, b_ref, w_ref, out_ref):
    a = acc_ref[...]
    ssum = a[0] + a[1] + g_ref[...]
    h = jnp.maximum(ssum * dinv_ref[...] + b_ref[...], 0.0)
    out_ref[...] = jnp.dot(h, w_ref[...],
                           preferred_element_type=jnp.float32) * dinv_ref[...]


def _tc2(acc, g, dinv, b, W2):
    return pl.pallas_call(
        _mid_body,
        grid=(_N // _R,),
        in_specs=[
            pl.BlockSpec((_NC, _R, _D), lambda j: (0, j, 0)),
            pl.BlockSpec((_R, _D), lambda j: (j, 0)),
            pl.BlockSpec((_R, 1), lambda j: (j, 0)),
            pl.BlockSpec((1, _D), lambda j: (0, 0)),
            pl.BlockSpec((_D, _D), lambda j: (0, 0)),
        ],
        out_specs=pl.BlockSpec((_R, _D), lambda j: (j, 0)),
        out_shape=jax.ShapeDtypeStruct((_N, _D), jnp.float32),
    )(acc, g, dinv, b, W2)


def _fin_body(acc_ref, g_ref, dinv_ref, b_ref, batch_ref, out_ref):
    j = pl.program_id(0)
    a = acc_ref[...]
    h = (a[0] + a[1] + g_ref[...]) * dinv_ref[...] + b_ref[...]
    bb = batch_ref[...]

    @pl.when(j == 0)
    def _():
        out_ref[...] = jnp.full((_G, _D), -jnp.inf, jnp.float32)

    for gi in range(_G):
        vals = jnp.where(bb == gi, h, -jnp.inf)
        m = jnp.max(vals, axis=0)
        out_ref[gi, :] = jnp.maximum(out_ref[gi, :], m)


def _tc3(acc, g, dinv, b, batch2d):
    return pl.pallas_call(
        _fin_body,
        grid=(_N // _R,),
        in_specs=[
            pl.BlockSpec((_NC, _R, _D), lambda j: (0, j, 0)),
            pl.BlockSpec((_R, _D), lambda j: (j, 0)),
            pl.BlockSpec((_R, 1), lambda j: (j, 0)),
            pl.BlockSpec((1, _D), lambda j: (0, 0)),
            pl.BlockSpec((_R, 1), lambda j: (j, 0)),
        ],
        out_specs=pl.BlockSpec((_G, _D), lambda j: (0, 0)),
        out_shape=jax.ShapeDtypeStruct((_G, _D), jnp.float32),
    )(acc, g, dinv, b, batch2d)


def kernel(x, edge_index, batch, W1, b1, W2, b2):
    npad = _EP - _E
    pad_src = (jnp.arange(npad, dtype=jnp.int32) % _N)
    pad_dst = _N + (jnp.arange(npad, dtype=jnp.int32) % (_NP - _N))
    srcp = jnp.concatenate([edge_index[0], pad_src]).reshape(_NW, _CHT, _K)
    dstp = jnp.concatenate([edge_index[1], pad_dst]).reshape(_NW, _CHT, _K)
    eidx = jnp.stack([srcp, dstp], axis=2).reshape(_NW, 2 * _CHT, _K)

    zeros_nd = jnp.zeros((_NP, _D), jnp.float32)
    ones_kd = jnp.ones((_K, _D), jnp.float32)

    degp = _deg_call()(eidx, ones_kd, zeros_nd)
    g1, dinv = _tc1(degp, x, W1)
    acc1 = _edge_call()(g1, eidx, zeros_nd)
    g2 = _tc2(acc1, g1, dinv, b1.reshape(1, _D), W2)
    acc2 = _edge_call()(g2, eidx, zeros_nd)
    out = _tc3(acc2, g2, dinv, b2.reshape(1, _D), batch.reshape(_N, 1))
    return out


# speedup vs baseline: 22.2693x; 1.1777x over previous
import functools

import jax
import jax.numpy as jnp
from jax import lax
from jax.experimental import pallas as pl
from jax.experimental.pallas import tpu as pltpu
from jax.experimental.pallas import tpu_sc as plsc

_N = 10000
_E = 320000
_D = 128
_G = 16
_NC = 2
_NS = 16
_NW = _NC * _NS
_K = 32
_EP = 327680
_CHT = 320
_SB = 4
_NIT = _CHT // (2 * _SB)
_NP = 10240
_RPT = _NP // _NS
_R = 1000


def _deg_body(eidx_hbm, ones_hbm, zeros_hbm, out_hbm, ib0, ib1, ones_v, acc,
              ssem, isem):
    c = lax.axis_index("c")
    s = lax.axis_index("s")
    w = c * _NS + s
    pltpu.sync_copy(zeros_hbm.at[pl.ds(s * _RPT, _RPT)],
                    acc.at[pl.ds(s * _RPT, _RPT)])
    pltpu.sync_copy(ones_hbm, ones_v)
    plsc.subcore_barrier()
    pltpu.async_copy(eidx_hbm.at[w, pl.ds(0, 2 * _SB)], ib0, isem)

    def body(j, carry):
        for p, ib, ibn in ((0, ib0, ib1), (1, ib1, ib0)):
            pltpu.make_async_copy(
                eidx_hbm.at[w, pl.ds(16 * j + 8 * p, 2 * _SB)], ib, isem).wait()

            if p == 0:
                @pl.when(j != 0)
                def _():
                    for t in range(_SB):
                        pltpu.make_async_copy(
                            ones_v, acc.at[ibn.at[2 * t + 1]], ssem).wait()
                pltpu.async_copy(
                    eidx_hbm.at[w, pl.ds(16 * j + 8, 2 * _SB)], ibn, isem)
            else:
                for t in range(_SB):
                    pltpu.make_async_copy(
                        ones_v, acc.at[ibn.at[2 * t + 1]], ssem).wait()

                @pl.when(j != _NIT - 1)
                def _():
                    pltpu.async_copy(
                        eidx_hbm.at[w, pl.ds(16 * j + 16, 2 * _SB)], ibn, isem)

            for t in range(_SB):
                pltpu.async_copy(
                    ones_v, acc.at[ib.at[2 * t + 1]], ssem, add=True)
        return carry

    lax.fori_loop(0, _NIT, body, 0)
    for t in range(_SB):
        pltpu.make_async_copy(ones_v, acc.at[ib1.at[2 * t + 1]], ssem).wait()
    plsc.subcore_barrier()
    pltpu.sync_copy(acc.at[pl.ds(s * _RPT, _RPT)],
                    out_hbm.at[c, pl.ds(s * _RPT, _RPT)])


@functools.lru_cache(maxsize=None)
def _deg_call():
    mesh = plsc.VectorSubcoreMesh(core_axis_name="c", subcore_axis_name="s")
    return pl.kernel(
        _deg_body,
        out_type=jax.ShapeDtypeStruct((_NC, _NP, _D), jnp.float32),
        mesh=mesh,
        scratch_types=[
            pltpu.VMEM((2 * _SB, _K), jnp.int32),
            pltpu.VMEM((2 * _SB, _K), jnp.int32),
            pltpu.VMEM((_K, _D), jnp.float32),
            pltpu.VMEM_SHARED((_NP, _D), jnp.float32),
            pltpu.SemaphoreType.DMA,
            pltpu.SemaphoreType.DMA,
        ],
    )


def _edge_body(g_hbm, eidx_hbm, zeros_hbm, out_hbm,
               ib0, ib1, r0, r1, r2, r3, r4, r5, r6, r7, acc,
               gsem, ssem, isem):
    c = lax.axis_index("c")
    s = lax.axis_index("s")
    w = c * _NS + s
    rows = (r0, r1, r2, r3, r4, r5, r6, r7)
    pltpu.sync_copy(zeros_hbm.at[pl.ds(s * _RPT, _RPT)],
                    acc.at[pl.ds(s * _RPT, _RPT)])
    plsc.subcore_barrier()

    pltpu.async_copy(eidx_hbm.at[w, pl.ds(0, 2 * _SB)], ib0, isem)

    def body(j, carry):
        for p, ib, ibn in ((0, ib0, ib1), (1, ib1, ib0)):
            bufs = rows[_SB * p:_SB * (p + 1)]
            obufs = rows[_SB * (1 - p):_SB * (2 - p)]

            pltpu.make_async_copy(
                eidx_hbm.at[w, pl.ds(16 * j + 8 * p, 2 * _SB)], ib, isem).wait()
            for t in range(_SB):
                pltpu.async_copy(g_hbm.at[ib.at[2 * t]], bufs[t], gsem)

            if p == 0:
                @pl.when(j != 0)
                def _():
                    for t in range(_SB):
                        pltpu.make_async_copy(
                            obufs[t], acc.at[ibn.at[2 * t + 1]], ssem).wait()
                pltpu.async_copy(
                    eidx_hbm.at[w, pl.ds(16 * j + 8, 2 * _SB)], ibn, isem)
            else:
                for t in range(_SB):
                    pltpu.make_async_copy(
                        obufs[t], acc.at[ibn.at[2 * t + 1]], ssem).wait()

                @pl.when(j != _NIT - 1)
                def _():
                    pltpu.async_copy(
                        eidx_hbm.at[w, pl.ds(16 * j + 16, 2 * _SB)], ibn, isem)

            for t in range(_SB):
                pltpu.make_async_copy(
                    g_hbm.at[ib.at[2 * t]], bufs[t], gsem).wait()
            for t in range(_SB):
                pltpu.async_copy(
                    bufs[t], acc.at[ib.at[2 * t + 1]], ssem, add=True)
        return carry

    lax.fori_loop(0, _NIT, body, 0)
    for t in range(_SB):
        pltpu.make_async_copy(
            rows[_SB + t], acc.at[ib1.at[2 * t + 1]], ssem).wait()
    plsc.subcore_barrier()
    pltpu.sync_copy(acc.at[pl.ds(s * _RPT, _RPT)],
                    out_hbm.at[c, pl.ds(s * _RPT, _RPT)])


@functools.lru_cache(maxsize=None)
def _edge_call():
    mesh = plsc.VectorSubcoreMesh(core_axis_name="c", subcore_axis_name="s")
    return pl.kernel(
        _edge_body,
        out_type=jax.ShapeDtypeStruct((_NC, _NP, _D), jnp.float32),
        mesh=mesh,
        scratch_types=[
            pltpu.VMEM((2 * _SB, _K), jnp.int32),
            pltpu.VMEM((2 * _SB, _K), jnp.int32),
        ] + [pltpu.VMEM((_K, _D), jnp.float32)] * 8 + [
            pltpu.VMEM_SHARED((_NP, _D), jnp.float32),
            pltpu.SemaphoreType.DMA,
            pltpu.SemaphoreType.DMA,
            pltpu.SemaphoreType.DMA,
        ],
    )


def _mm1_body(d_ref, x_ref, w_ref, g_ref, dinv_ref):
    db = d_ref[...]
    deg = db[0, :, :1] + db[1, :, :1] + 1.0
    dinv = lax.rsqrt(deg)
    hw = jnp.dot(x_ref[...], w_ref[...], preferred_element_type=jnp.float32)
    g_ref[...] = hw * dinv
    dinv_ref[...] = dinv


def _tc1(degp, x, W1):
    return pl.pallas_call(
        _mm1_body,
        grid=(_N // _R,),
        in_specs=[
            pl.BlockSpec((_NC, _R, _D), lambda j: (0, j, 0)),
            pl.BlockSpec((_R, _D), lambda j: (j, 0)),
            pl.BlockSpec((_D, _D), lambda j: (0, 0)),
        ],
        out_specs=[
            pl.BlockSpec((_R, _D), lambda j: (j, 0)),
            pl.BlockSpec((_R, 1), lambda j: (j, 0)),
        ],
        out_shape=[
            jax.ShapeDtypeStruct((_N, _D), jnp.float32),
            jax.ShapeDtypeStruct((_N, 1), jnp.float32),
        ],
    )(degp, x, W1)


def _mid_body(acc_ref, g_ref, dinv_ref, b_ref, w_ref, out_ref):
    a = acc_ref[...]
    ssum = a[0] + a[1] + g_ref[...]
    h = jnp.maximum(ssum * dinv_ref[...] + b_ref[...], 0.0)
    out_ref[...] = jnp.dot(h, w_ref[...],
                           preferred_element_type=jnp.float32) * dinv_ref[...]


def _tc2(acc, g, dinv, b, W2):
    return pl.pallas_call(
        _mid_body,
        grid=(_N // _R,),
        in_specs=[
            pl.BlockSpec((_NC, _R, _D), lambda j: (0, j, 0)),
            pl.BlockSpec((_R, _D), lambda j: (j, 0)),
            pl.BlockSpec((_R, 1), lambda j: (j, 0)),
            pl.BlockSpec((1, _D), lambda j: (0, 0)),
            pl.BlockSpec((_D, _D), lambda j: (0, 0)),
        ],
        out_specs=pl.BlockSpec((_R, _D), lambda j: (j, 0)),
        out_shape=jax.ShapeDtypeStruct((_N, _D), jnp.float32),
    )(acc, g, dinv, b, W2)


def _fin_body(acc_ref, g_ref, dinv_ref, b_ref, batch_ref, out_ref):
    j = pl.program_id(0)
    a = acc_ref[...]
    h = (a[0] + a[1] + g_ref[...]) * dinv_ref[...] + b_ref[...]
    bb = batch_ref[...]

    @pl.when(j == 0)
    def _():
        out_ref[...] = jnp.full((_G, _D), -jnp.inf, jnp.float32)

    for gi in range(_G):
        vals = jnp.where(bb == gi, h, -jnp.inf)
        m = jnp.max(vals, axis=0)
        out_ref[gi, :] = jnp.maximum(out_ref[gi, :], m)


def _tc3(acc, g, dinv, b, batch2d):
    return pl.pallas_call(
        _fin_body,
        grid=(_N // _R,),
        in_specs=[
            pl.BlockSpec((_NC, _R, _D), lambda j: (0, j, 0)),
            pl.BlockSpec((_R, _D), lambda j: (j, 0)),
            pl.BlockSpec((_R, 1), lambda j: (j, 0)),
            pl.BlockSpec((1, _D), lambda j: (0, 0)),
            pl.BlockSpec((_R, 1), lambda j: (j, 0)),
        ],
        out_specs=pl.BlockSpec((_G, _D), lambda j: (0, 0)),
        out_shape=jax.ShapeDtypeStruct((_G, _D), jnp.float32),
    )(acc, g, dinv, b, batch2d)


def kernel(x, edge_index, batch, W1, b1, W2, b2):
    npad = _EP - _E
    pad_src = (jnp.arange(npad, dtype=jnp.int32) % _N)
    pad_dst = _N + (jnp.arange(npad, dtype=jnp.int32) % (_NP - _N))
    srcp = jnp.concatenate([edge_index[0], pad_src]).reshape(_NW, _CHT, _K)
    dstp = jnp.concatenate([edge_index[1], pad_dst]).reshape(_NW, _CHT, _K)
    eidx = jnp.stack([srcp, dstp], axis=2).reshape(_NW, 2 * _CHT, _K)

    zeros_nd = jnp.zeros((_NP, _D), jnp.float32)
    ones_kd = jnp.ones((_K, _D), jnp.float32)

    degp = _deg_call()(eidx, ones_kd, zeros_nd)
    g1, dinv = _tc1(degp, x, W1)
    acc1 = _edge_call()(g1, eidx, zeros_nd)
    g2 = _tc2(acc1, g1, dinv, b1.reshape(1, _D), W2)
    acc2 = _edge_call()(g2, eidx, zeros_nd)
    out = _tc3(acc2, g2, dinv, b2.reshape(1, _D), batch.reshape(_N, 1))
    return out


# speedup vs baseline: 23.0417x; 1.0347x over previous
import functools

import jax
import jax.numpy as jnp
from jax import lax
from jax.experimental import pallas as pl
from jax.experimental.pallas import tpu as pltpu
from jax.experimental.pallas import tpu_sc as plsc

_N = 10000
_E = 320000
_D = 128
_G = 16
_NC = 2
_NS = 16
_NW = _NC * _NS
_K = 32
_EP = 327680
_CHT = 320
_SB = 4
_NIT = _CHT // (2 * _SB)
_NP = 10240
_RPT = _NP // _NS
_R = 1000


def _deg_body(eidx_hbm, ones_hbm, zeros_hbm, out_hbm, ib0, ib1, ones_v, acc,
              ssem, isem):
    c = lax.axis_index("c")
    s = lax.axis_index("s")
    w = c * _NS + s
    pltpu.sync_copy(zeros_hbm.at[pl.ds(s * _RPT, _RPT)],
                    acc.at[pl.ds(s * _RPT, _RPT)])
    pltpu.sync_copy(ones_hbm, ones_v)
    plsc.subcore_barrier()
    pltpu.async_copy(eidx_hbm.at[w, pl.ds(0, 2 * _SB)], ib0, isem)

    def body(j, carry):
        for p, ib, ibn in ((0, ib0, ib1), (1, ib1, ib0)):
            pltpu.make_async_copy(
                eidx_hbm.at[w, pl.ds(16 * j + 8 * p, 2 * _SB)], ib, isem).wait()

            if p == 0:
                @pl.when(j != 0)
                def _():
                    for t in range(_SB):
                        pltpu.make_async_copy(
                            ones_v, acc.at[ibn.at[2 * t + 1]], ssem).wait()
                pltpu.async_copy(
                    eidx_hbm.at[w, pl.ds(16 * j + 8, 2 * _SB)], ibn, isem)
            else:
                for t in range(_SB):
                    pltpu.make_async_copy(
                        ones_v, acc.at[ibn.at[2 * t + 1]], ssem).wait()

                @pl.when(j != _NIT - 1)
                def _():
                    pltpu.async_copy(
                        eidx_hbm.at[w, pl.ds(16 * j + 16, 2 * _SB)], ibn, isem)

            for t in range(_SB):
                pltpu.async_copy(
                    ones_v, acc.at[ib.at[2 * t + 1]], ssem, add=True)
        return carry

    lax.fori_loop(0, _NIT, body, 0)
    for t in range(_SB):
        pltpu.make_async_copy(ones_v, acc.at[ib1.at[2 * t + 1]], ssem).wait()
    plsc.subcore_barrier()
    pltpu.sync_copy(acc.at[pl.ds(s * _RPT, _RPT)],
                    out_hbm.at[c, pl.ds(s * _RPT, _RPT)])


@functools.lru_cache(maxsize=None)
def _deg_call():
    mesh = plsc.VectorSubcoreMesh(core_axis_name="c", subcore_axis_name="s")
    return pl.kernel(
        _deg_body,
        out_type=jax.ShapeDtypeStruct((_NC, _NP), jnp.float32),
        mesh=mesh,
        scratch_types=[
            pltpu.VMEM((2 * _SB, _K), jnp.int32),
            pltpu.VMEM((2 * _SB, _K), jnp.int32),
            pltpu.VMEM((_K,), jnp.float32),
            pltpu.VMEM_SHARED((_NP,), jnp.float32),
            pltpu.SemaphoreType.DMA,
            pltpu.SemaphoreType.DMA,
        ],
    )


def _edge_body(g_hbm, eidx_hbm, zeros_hbm, out_hbm,
               ib0, ib1, r0, r1, r2, r3, r4, r5, r6, r7, acc,
               gsem, ssem, isem):
    c = lax.axis_index("c")
    s = lax.axis_index("s")
    w = c * _NS + s
    rows = (r0, r1, r2, r3, r4, r5, r6, r7)
    pltpu.sync_copy(zeros_hbm.at[pl.ds(s * _RPT, _RPT)],
                    acc.at[pl.ds(s * _RPT, _RPT)])
    plsc.subcore_barrier()

    pltpu.async_copy(eidx_hbm.at[w, pl.ds(0, 2 * _SB)], ib0, isem)

    def body(j, carry):
        for p, ib, ibn in ((0, ib0, ib1), (1, ib1, ib0)):
            bufs = rows[_SB * p:_SB * (p + 1)]
            obufs = rows[_SB * (1 - p):_SB * (2 - p)]

            pltpu.make_async_copy(
                eidx_hbm.at[w, pl.ds(16 * j + 8 * p, 2 * _SB)], ib, isem).wait()
            for t in range(_SB):
                pltpu.async_copy(g_hbm.at[ib.at[2 * t]], bufs[t], gsem)

            if p == 0:
                @pl.when(j != 0)
                def _():
                    for t in range(_SB):
                        pltpu.make_async_copy(
                            obufs[t], acc.at[ibn.at[2 * t + 1]], ssem).wait()
                pltpu.async_copy(
                    eidx_hbm.at[w, pl.ds(16 * j + 8, 2 * _SB)], ibn, isem)
            else:
                for t in range(_SB):
                    pltpu.make_async_copy(
                        obufs[t], acc.at[ibn.at[2 * t + 1]], ssem).wait()

                @pl.when(j != _NIT - 1)
                def _():
                    pltpu.async_copy(
                        eidx_hbm.at[w, pl.ds(16 * j + 16, 2 * _SB)], ibn, isem)

            for t in range(_SB):
                pltpu.make_async_copy(
                    g_hbm.at[ib.at[2 * t]], bufs[t], gsem).wait()
            for t in range(_SB):
                pltpu.async_copy(
                    bufs[t], acc.at[ib.at[2 * t + 1]], ssem, add=True)
        return carry

    lax.fori_loop(0, _NIT, body, 0)
    for t in range(_SB):
        pltpu.make_async_copy(
            rows[_SB + t], acc.at[ib1.at[2 * t + 1]], ssem).wait()
    plsc.subcore_barrier()
    pltpu.sync_copy(acc.at[pl.ds(s * _RPT, _RPT)],
                    out_hbm.at[c, pl.ds(s * _RPT, _RPT)])


@functools.lru_cache(maxsize=None)
def _edge_call():
    mesh = plsc.VectorSubcoreMesh(core_axis_name="c", subcore_axis_name="s")
    return pl.kernel(
        _edge_body,
        out_type=jax.ShapeDtypeStruct((_NC, _NP, _D), jnp.float32),
        mesh=mesh,
        scratch_types=[
            pltpu.VMEM((2 * _SB, _K), jnp.int32),
            pltpu.VMEM((2 * _SB, _K), jnp.int32),
        ] + [pltpu.VMEM((_K, _D), jnp.float32)] * 8 + [
            pltpu.VMEM_SHARED((_NP, _D), jnp.float32),
            pltpu.SemaphoreType.DMA,
            pltpu.SemaphoreType.DMA,
            pltpu.SemaphoreType.DMA,
        ],
    )


def _mm1_body(d0_ref, d1_ref, x_ref, w_ref, g_ref, dinv_ref):
    deg = d0_ref[...] + d1_ref[...] + 1.0
    dinv = lax.rsqrt(deg)
    hw = jnp.dot(x_ref[...], w_ref[...], preferred_element_type=jnp.float32)
    g_ref[...] = hw * dinv
    dinv_ref[...] = dinv


def _tc1(d0, d1, x, W1):
    return pl.pallas_call(
        _mm1_body,
        grid=(_N // _R,),
        in_specs=[
            pl.BlockSpec((_R, 1), lambda j: (j, 0)),
            pl.BlockSpec((_R, 1), lambda j: (j, 0)),
            pl.BlockSpec((_R, _D), lambda j: (j, 0)),
            pl.BlockSpec((_D, _D), lambda j: (0, 0)),
        ],
        out_specs=[
            pl.BlockSpec((_R, _D), lambda j: (j, 0)),
            pl.BlockSpec((_R, 1), lambda j: (j, 0)),
        ],
        out_shape=[
            jax.ShapeDtypeStruct((_N, _D), jnp.float32),
            jax.ShapeDtypeStruct((_N, 1), jnp.float32),
        ],
    )(d0, d1, x, W1)


def _mid_body(acc_ref, g_ref, dinv_ref, b_ref, w_ref, out_ref):
    a = acc_ref[...]
    ssum = a[0] + a[1] + g_ref[...]
    h = jnp.maximum(ssum * dinv_ref[...] + b_ref[...], 0.0)
    out_ref[...] = jnp.dot(h, w_ref[...],
                           preferred_element_type=jnp.float32) * dinv_ref[...]


def _tc2(acc, g, dinv, b, W2):
    return pl.pallas_call(
        _mid_body,
        grid=(_N // _R,),
        in_specs=[
            pl.BlockSpec((_NC, _R, _D), lambda j: (0, j, 0)),
            pl.BlockSpec((_R, _D), lambda j: (j, 0)),
            pl.BlockSpec((_R, 1), lambda j: (j, 0)),
            pl.BlockSpec((1, _D), lambda j: (0, 0)),
            pl.BlockSpec((_D, _D), lambda j: (0, 0)),
        ],
        out_specs=pl.BlockSpec((_R, _D), lambda j: (j, 0)),
        out_shape=jax.ShapeDtypeStruct((_N, _D), jnp.float32),
    )(acc, g, dinv, b, W2)


def _fin_body(acc_ref, g_ref, dinv_ref, b_ref, batch_ref, out_ref):
    j = pl.program_id(0)
    a = acc_ref[...]
    h = (a[0] + a[1] + g_ref[...]) * dinv_ref[...] + b_ref[...]
    bb = batch_ref[...]

    @pl.when(j == 0)
    def _():
        out_ref[...] = jnp.full((_G, _D), -jnp.inf, jnp.float32)

    for gi in range(_G):
        vals = jnp.where(bb == gi, h, -jnp.inf)
        m = jnp.max(vals, axis=0)
        out_ref[gi, :] = jnp.maximum(out_ref[gi, :], m)


def _tc3(acc, g, dinv, b, batch2d):
    return pl.pallas_call(
        _fin_body,
        grid=(_N // _R,),
        in_specs=[
            pl.BlockSpec((_NC, _R, _D), lambda j: (0, j, 0)),
            pl.BlockSpec((_R, _D), lambda j: (j, 0)),
            pl.BlockSpec((_R, 1), lambda j: (j, 0)),
            pl.BlockSpec((1, _D), lambda j: (0, 0)),
            pl.BlockSpec((_R, 1), lambda j: (j, 0)),
        ],
        out_specs=pl.BlockSpec((_G, _D), lambda j: (0, 0)),
        out_shape=jax.ShapeDtypeStruct((_G, _D), jnp.float32),
    )(acc, g, dinv, b, batch2d)


def kernel(x, edge_index, batch, W1, b1, W2, b2):
    npad = _EP - _E
    pad_src = (jnp.arange(npad, dtype=jnp.int32) % _N)
    pad_dst = _N + (jnp.arange(npad, dtype=jnp.int32) % (_NP - _N))
    srcp = jnp.concatenate([edge_index[0], pad_src]).reshape(_NW, _CHT, _K)
    dstp = jnp.concatenate([edge_index[1], pad_dst]).reshape(_NW, _CHT, _K)
    eidx = jnp.stack([srcp, dstp], axis=2).reshape(_NW, 2 * _CHT, _K)

    zeros_nd = jnp.zeros((_NP, _D), jnp.float32)
    zeros_n1 = jnp.zeros((_NP,), jnp.float32)
    ones_k1 = jnp.ones((_K,), jnp.float32)

    degp = _deg_call()(eidx, ones_k1, zeros_n1)
    d0 = degp[0].reshape(_NP, 1)
    d1 = degp[1].reshape(_NP, 1)
    g1, dinv = _tc1(d0, d1, x, W1)
    acc1 = _edge_call()(g1, eidx, zeros_nd)
    g2 = _tc2(acc1, g1, dinv, b1.reshape(1, _D), W2)
    acc2 = _edge_call()(g2, eidx, zeros_nd)
    out = _tc3(acc2, g2, dinv, b2.reshape(1, _D), batch.reshape(_N, 1))
    return out


# speedup vs baseline: 28.0207x; 1.2161x over previous
import functools

import jax
import jax.numpy as jnp
from jax import lax
from jax.experimental import pallas as pl
from jax.experimental.pallas import tpu as pltpu
from jax.experimental.pallas import tpu_sc as plsc

_N = 10000
_E = 320000
_D = 128
_G = 16
_NC = 2
_NS = 16
_NW = _NC * _NS
_K = 40
_EP = 327680
_CHT = 256
_SB = 4
_NIT = _CHT // (2 * _SB)
_KD = 128
_CHD = 80
_SBD = 8
_NITD = _CHD // (2 * _SBD)
_NP = 10240
_RPT = _NP // _NS
_R = 1000


def _deg_body(didx_hbm, ones_hbm, zeros_hbm, out_hbm, ib0, ib1, ones_v, acc,
              ssem, isem):
    c = lax.axis_index("c")
    s = lax.axis_index("s")
    w = c * _NS + s
    pltpu.sync_copy(zeros_hbm.at[pl.ds(s * _RPT, _RPT)],
                    acc.at[pl.ds(s * _RPT, _RPT)])
    pltpu.sync_copy(ones_hbm, ones_v)
    plsc.subcore_barrier()
    pltpu.async_copy(didx_hbm.at[w, pl.ds(0, _SBD)], ib0, isem)

    def body(j, carry):
        for p, ib, ibn in ((0, ib0, ib1), (1, ib1, ib0)):
            pltpu.make_async_copy(
                didx_hbm.at[w, pl.ds(2 * _SBD * j + _SBD * p, _SBD)], ib,
                isem).wait()

            if p == 0:
                @pl.when(j != 0)
                def _():
                    for t in range(_SBD):
                        pltpu.make_async_copy(
                            ones_v, acc.at[ibn.at[t]], ssem).wait()
                pltpu.async_copy(
                    didx_hbm.at[w, pl.ds(2 * _SBD * j + _SBD, _SBD)], ibn, isem)
            else:
                for t in range(_SBD):
                    pltpu.make_async_copy(
                        ones_v, acc.at[ibn.at[t]], ssem).wait()

                @pl.when(j != _NITD - 1)
                def _():
                    pltpu.async_copy(
                        didx_hbm.at[w, pl.ds(2 * _SBD * j + 2 * _SBD, _SBD)],
                        ibn, isem)

            for t in range(_SBD):
                pltpu.async_copy(ones_v, acc.at[ib.at[t]], ssem, add=True)
        return carry

    lax.fori_loop(0, _NITD, body, 0)
    for t in range(_SBD):
        pltpu.make_async_copy(ones_v, acc.at[ib1.at[t]], ssem).wait()
    plsc.subcore_barrier()
    pltpu.sync_copy(acc.at[pl.ds(s * _RPT, _RPT)],
                    out_hbm.at[c, pl.ds(s * _RPT, _RPT)])


@functools.lru_cache(maxsize=None)
def _deg_call():
    mesh = plsc.VectorSubcoreMesh(core_axis_name="c", subcore_axis_name="s")
    return pl.kernel(
        _deg_body,
        out_type=jax.ShapeDtypeStruct((_NC, _NP), jnp.float32),
        mesh=mesh,
        scratch_types=[
            pltpu.VMEM((_SBD, _KD), jnp.int32),
            pltpu.VMEM((_SBD, _KD), jnp.int32),
            pltpu.VMEM((_KD,), jnp.float32),
            pltpu.VMEM_SHARED((_NP,), jnp.float32),
            pltpu.SemaphoreType.DMA,
            pltpu.SemaphoreType.DMA,
        ],
    )


def _edge_body(g_hbm, eidx_hbm, zeros_hbm, out_hbm,
               ib0, ib1, r0, r1, r2, r3, r4, r5, r6, r7, acc,
               gsem, ssem, isem):
    c = lax.axis_index("c")
    s = lax.axis_index("s")
    w = c * _NS + s
    rows = (r0, r1, r2, r3, r4, r5, r6, r7)
    pltpu.sync_copy(zeros_hbm.at[pl.ds(s * _RPT, _RPT)],
                    acc.at[pl.ds(s * _RPT, _RPT)])
    plsc.subcore_barrier()

    pltpu.async_copy(eidx_hbm.at[w, pl.ds(0, 2 * _SB)], ib0, isem)

    def body(j, carry):
        for p, ib, ibn in ((0, ib0, ib1), (1, ib1, ib0)):
            bufs = rows[_SB * p:_SB * (p + 1)]
            obufs = rows[_SB * (1 - p):_SB * (2 - p)]

            pltpu.make_async_copy(
                eidx_hbm.at[w, pl.ds(16 * j + 8 * p, 2 * _SB)], ib, isem).wait()
            for t in range(_SB):
                pltpu.async_copy(g_hbm.at[ib.at[2 * t]], bufs[t], gsem)

            if p == 0:
                @pl.when(j != 0)
                def _():
                    for t in range(_SB):
                        pltpu.make_async_copy(
                            obufs[t], acc.at[ibn.at[2 * t + 1]], ssem).wait()
                pltpu.async_copy(
                    eidx_hbm.at[w, pl.ds(16 * j + 8, 2 * _SB)], ibn, isem)
            else:
                for t in range(_SB):
                    pltpu.make_async_copy(
                        obufs[t], acc.at[ibn.at[2 * t + 1]], ssem).wait()

                @pl.when(j != _NIT - 1)
                def _():
                    pltpu.async_copy(
                        eidx_hbm.at[w, pl.ds(16 * j + 16, 2 * _SB)], ibn, isem)

            for t in range(_SB):
                pltpu.make_async_copy(
                    g_hbm.at[ib.at[2 * t]], bufs[t], gsem).wait()
            for t in range(_SB):
                pltpu.async_copy(
                    bufs[t], acc.at[ib.at[2 * t + 1]], ssem, add=True)
        return carry

    lax.fori_loop(0, _NIT, body, 0)
    for t in range(_SB):
        pltpu.make_async_copy(
            rows[_SB + t], acc.at[ib1.at[2 * t + 1]], ssem).wait()
    plsc.subcore_barrier()
    pltpu.sync_copy(acc.at[pl.ds(s * _RPT, _RPT)],
                    out_hbm.at[c, pl.ds(s * _RPT, _RPT)])


@functools.lru_cache(maxsize=None)
def _edge_call():
    mesh = plsc.VectorSubcoreMesh(core_axis_name="c", subcore_axis_name="s")
    return pl.kernel(
        _edge_body,
        out_type=jax.ShapeDtypeStruct((_NC, _NP, _D), jnp.float32),
        mesh=mesh,
        scratch_types=[
            pltpu.VMEM((2 * _SB, _K), jnp.int32),
            pltpu.VMEM((2 * _SB, _K), jnp.int32),
        ] + [pltpu.VMEM((_K, _D), jnp.float32)] * 8 + [
            pltpu.VMEM_SHARED((_NP, _D), jnp.float32),
            pltpu.SemaphoreType.DMA,
            pltpu.SemaphoreType.DMA,
            pltpu.SemaphoreType.DMA,
        ],
    )


def _mm1_body(d0_ref, d1_ref, x_ref, w_ref, g_ref, dinv_ref):
    deg = d0_ref[...] + d1_ref[...] + 1.0
    dinv = lax.rsqrt(deg)
    hw = jnp.dot(x_ref[...], w_ref[...], preferred_element_type=jnp.float32)
    g_ref[...] = hw * dinv
    dinv_ref[...] = dinv


def _tc1(d0, d1, x, W1):
    return pl.pallas_call(
        _mm1_body,
        grid=(_N // _R,),
        in_specs=[
            pl.BlockSpec((_R, 1), lambda j: (j, 0)),
            pl.BlockSpec((_R, 1), lambda j: (j, 0)),
            pl.BlockSpec((_R, _D), lambda j: (j, 0)),
            pl.BlockSpec((_D, _D), lambda j: (0, 0)),
        ],
        out_specs=[
            pl.BlockSpec((_R, _D), lambda j: (j, 0)),
            pl.BlockSpec((_R, 1), lambda j: (j, 0)),
        ],
        out_shape=[
            jax.ShapeDtypeStruct((_N, _D), jnp.float32),
            jax.ShapeDtypeStruct((_N, 1), jnp.float32),
        ],
    )(d0, d1, x, W1)


def _mid_body(acc_ref, g_ref, dinv_ref, b_ref, w_ref, out_ref):
    a = acc_ref[...]
    ssum = a[0] + a[1] + g_ref[...]
    h = jnp.maximum(ssum * dinv_ref[...] + b_ref[...], 0.0)
    out_ref[...] = jnp.dot(h, w_ref[...],
                           preferred_element_type=jnp.float32) * dinv_ref[...]


def _tc2(acc, g, dinv, b, W2):
    return pl.pallas_call(
        _mid_body,
        grid=(_N // _R,),
        in_specs=[
            pl.BlockSpec((_NC, _R, _D), lambda j: (0, j, 0)),
            pl.BlockSpec((_R, _D), lambda j: (j, 0)),
            pl.BlockSpec((_R, 1), lambda j: (j, 0)),
            pl.BlockSpec((1, _D), lambda j: (0, 0)),
            pl.BlockSpec((_D, _D), lambda j: (0, 0)),
        ],
        out_specs=pl.BlockSpec((_R, _D), lambda j: (j, 0)),
        out_shape=jax.ShapeDtypeStruct((_N, _D), jnp.float32),
    )(acc, g, dinv, b, W2)


def _fin_body(acc_ref, g_ref, dinv_ref, b_ref, batch_ref, out_ref):
    j = pl.program_id(0)
    a = acc_ref[...]
    h = (a[0] + a[1] + g_ref[...]) * dinv_ref[...] + b_ref[...]
    bb = batch_ref[...]

    @pl.when(j == 0)
    def _():
        out_ref[...] = jnp.full((_G, _D), -jnp.inf, jnp.float32)

    for gi in range(_G):
        vals = jnp.where(bb == gi, h, -jnp.inf)
        m = jnp.max(vals, axis=0)
        out_ref[gi, :] = jnp.maximum(out_ref[gi, :], m)


def _tc3(acc, g, dinv, b, batch2d):
    return pl.pallas_call(
        _fin_body,
        grid=(_N // _R,),
        in_specs=[
            pl.BlockSpec((_NC, _R, _D), lambda j: (0, j, 0)),
            pl.BlockSpec((_R, _D), lambda j: (j, 0)),
            pl.BlockSpec((_R, 1), lambda j: (j, 0)),
            pl.BlockSpec((1, _D), lambda j: (0, 0)),
            pl.BlockSpec((_R, 1), lambda j: (j, 0)),
        ],
        out_specs=pl.BlockSpec((_G, _D), lambda j: (0, 0)),
        out_shape=jax.ShapeDtypeStruct((_G, _D), jnp.float32),
    )(acc, g, dinv, b, batch2d)


def kernel(x, edge_index, batch, W1, b1, W2, b2):
    npad = _EP - _E
    pad_src = (jnp.arange(npad, dtype=jnp.int32) % _N)
    pad_dst = _N + (jnp.arange(npad, dtype=jnp.int32) % (_NP - _N))
    srcf = jnp.concatenate([edge_index[0], pad_src])
    dstf = jnp.concatenate([edge_index[1], pad_dst])
    srcp = srcf.reshape(_NW, _CHT, _K)
    dstp = dstf.reshape(_NW, _CHT, _K)
    eidx = jnp.stack([srcp, dstp], axis=2).reshape(_NW, 2 * _CHT, _K)
    didx = dstf.reshape(_NW, _CHD, _KD)

    zeros_nd = jnp.zeros((_NP, _D), jnp.float32)
    zeros_n1 = jnp.zeros((_NP,), jnp.float32)
    ones_k1 = jnp.ones((_KD,), jnp.float32)

    degp = _deg_call()(didx, ones_k1, zeros_n1)
    d0 = degp[0].reshape(_NP, 1)
    d1 = degp[1].reshape(_NP, 1)
    g1, dinv = _tc1(d0, d1, x, W1)
    acc1 = _edge_call()(g1, eidx, zeros_nd)
    g2 = _tc2(acc1, g1, dinv, b1.reshape(1, _D), W2)
    acc2 = _edge_call()(g2, eidx, zeros_nd)
    out = _tc3(acc2, g2, dinv, b2.reshape(1, _D), batch.reshape(_N, 1))
    return out


# speedup vs baseline: 28.8574x; 1.0299x over previous
import functools

import jax
import jax.numpy as jnp
from jax import lax
from jax.experimental import pallas as pl
from jax.experimental.pallas import tpu as pltpu
from jax.experimental.pallas import tpu_sc as plsc

_N = 10000
_E = 320000
_D = 128
_G = 16
_NC = 2
_NS = 16
_NW = _NC * _NS
_K = 80
_EP = 327680
_CHT = 128
_NIT = _CHT // 8
_KD = 128
_CHD = 80
_SBD = 8
_NITD = _CHD // (2 * _SBD)
_NP = 10240
_RPT = _NP // _NS
_R = 1000


def _deg_body(didx_hbm, ones_hbm, zeros_hbm, out_hbm, ib0, ib1, ones_v, acc,
              ssem, isem):
    c = lax.axis_index("c")
    s = lax.axis_index("s")
    w = c * _NS + s
    pltpu.sync_copy(zeros_hbm.at[pl.ds(s * _RPT, _RPT)],
                    acc.at[pl.ds(s * _RPT, _RPT)])
    pltpu.sync_copy(ones_hbm, ones_v)
    plsc.subcore_barrier()
    pltpu.async_copy(didx_hbm.at[w, pl.ds(0, _SBD)], ib0, isem)

    def body(j, carry):
        for p, ib, ibn in ((0, ib0, ib1), (1, ib1, ib0)):
            pltpu.make_async_copy(
                didx_hbm.at[w, pl.ds(2 * _SBD * j + _SBD * p, _SBD)], ib,
                isem).wait()

            if p == 0:
                @pl.when(j != 0)
                def _():
                    for t in range(_SBD):
                        pltpu.make_async_copy(
                            ones_v, acc.at[ibn.at[t]], ssem).wait()
                pltpu.async_copy(
                    didx_hbm.at[w, pl.ds(2 * _SBD * j + _SBD, _SBD)], ibn, isem)
            else:
                for t in range(_SBD):
                    pltpu.make_async_copy(
                        ones_v, acc.at[ibn.at[t]], ssem).wait()

                @pl.when(j != _NITD - 1)
                def _():
                    pltpu.async_copy(
                        didx_hbm.at[w, pl.ds(2 * _SBD * j + 2 * _SBD, _SBD)],
                        ibn, isem)

            for t in range(_SBD):
                pltpu.async_copy(ones_v, acc.at[ib.at[t]], ssem, add=True)
        return carry

    lax.fori_loop(0, _NITD, body, 0)
    for t in range(_SBD):
        pltpu.make_async_copy(ones_v, acc.at[ib1.at[t]], ssem).wait()
    plsc.subcore_barrier()
    pltpu.sync_copy(acc.at[pl.ds(s * _RPT, _RPT)],
                    out_hbm.at[c, pl.ds(s * _RPT, _RPT)])


@functools.lru_cache(maxsize=None)
def _deg_call():
    mesh = plsc.VectorSubcoreMesh(core_axis_name="c", subcore_axis_name="s")
    return pl.kernel(
        _deg_body,
        out_type=jax.ShapeDtypeStruct((_NC, _NP), jnp.float32),
        mesh=mesh,
        scratch_types=[
            pltpu.VMEM((_SBD, _KD), jnp.int32),
            pltpu.VMEM((_SBD, _KD), jnp.int32),
            pltpu.VMEM((_KD,), jnp.float32),
            pltpu.VMEM_SHARED((_NP,), jnp.float32),
            pltpu.SemaphoreType.DMA,
            pltpu.SemaphoreType.DMA,
        ],
    )


def _edge_body(g_hbm, eidx_hbm, zeros_hbm, out_hbm,
               ib0, ib1, r0, r1, r2, r3, acc,
               gsem, ssem, isem):
    c = lax.axis_index("c")
    s = lax.axis_index("s")
    w = c * _NS + s
    rows = (r0, r1, r2, r3)
    pltpu.sync_copy(zeros_hbm.at[pl.ds(s * _RPT, _RPT)],
                    acc.at[pl.ds(s * _RPT, _RPT)])
    plsc.subcore_barrier()

    pltpu.async_copy(eidx_hbm.at[w, pl.ds(0, 8)], ib0, isem)

    def body(j, carry):
        pltpu.make_async_copy(
            eidx_hbm.at[w, pl.ds(16 * j, 8)], ib0, isem).wait()
        for t in (0, 1):
            pltpu.async_copy(g_hbm.at[ib0.at[2 * t]], rows[t], gsem)

        @pl.when(j != 0)
        def _():
            for t in (2, 3):
                pltpu.make_async_copy(
                    rows[t], acc.at[ib1.at[2 * t + 1]], ssem).wait()
        pltpu.async_copy(eidx_hbm.at[w, pl.ds(16 * j + 8, 8)], ib1, isem)
        for t in (0, 1):
            pltpu.make_async_copy(g_hbm.at[ib0.at[2 * t]], rows[t], gsem).wait()
        for t in (0, 1):
            pltpu.async_copy(rows[t], acc.at[ib0.at[2 * t + 1]], ssem, add=True)

        for t in (2, 3):
            pltpu.async_copy(g_hbm.at[ib0.at[2 * t]], rows[t], gsem)
        for t in (0, 1):
            pltpu.make_async_copy(
                rows[t], acc.at[ib0.at[2 * t + 1]], ssem).wait()
        for t in (2, 3):
            pltpu.make_async_copy(g_hbm.at[ib0.at[2 * t]], rows[t], gsem).wait()
        for t in (2, 3):
            pltpu.async_copy(rows[t], acc.at[ib0.at[2 * t + 1]], ssem, add=True)

        pltpu.make_async_copy(
            eidx_hbm.at[w, pl.ds(16 * j + 8, 8)], ib1, isem).wait()
        for t in (0, 1):
            pltpu.async_copy(g_hbm.at[ib1.at[2 * t]], rows[t], gsem)
        for t in (2, 3):
            pltpu.make_async_copy(
                rows[t], acc.at[ib0.at[2 * t + 1]], ssem).wait()

        @pl.when(j != _NIT - 1)
        def _():
            pltpu.async_copy(eidx_hbm.at[w, pl.ds(16 * j + 16, 8)], ib0, isem)
        for t in (0, 1):
            pltpu.make_async_copy(g_hbm.at[ib1.at[2 * t]], rows[t], gsem).wait()
        for t in (0, 1):
            pltpu.async_copy(rows[t], acc.at[ib1.at[2 * t + 1]], ssem, add=True)

        for t in (2, 3):
            pltpu.async_copy(g_hbm.at[ib1.at[2 * t]], rows[t], gsem)
        for t in (0, 1):
            pltpu.make_async_copy(
                rows[t], acc.at[ib1.at[2 * t + 1]], ssem).wait()
        for t in (2, 3):
            pltpu.make_async_copy(g_hbm.at[ib1.at[2 * t]], rows[t], gsem).wait()
        for t in (2, 3):
            pltpu.async_copy(rows[t], acc.at[ib1.at[2 * t + 1]], ssem, add=True)
        return carry

    lax.fori_loop(0, _NIT, body, 0)
    for t in (2, 3):
        pltpu.make_async_copy(
            rows[t], acc.at[ib1.at[2 * t + 1]], ssem).wait()
    plsc.subcore_barrier()
    pltpu.sync_copy(acc.at[pl.ds(s * _RPT, _RPT)],
                    out_hbm.at[c, pl.ds(s * _RPT, _RPT)])


@functools.lru_cache(maxsize=None)
def _edge_call():
    mesh = plsc.VectorSubcoreMesh(core_axis_name="c", subcore_axis_name="s")
    return pl.kernel(
        _edge_body,
        out_type=jax.ShapeDtypeStruct((_NC, _NP, _D), jnp.float32),
        mesh=mesh,
        scratch_types=[
            pltpu.VMEM((8, _K), jnp.int32),
            pltpu.VMEM((8, _K), jnp.int32),
        ] + [pltpu.VMEM((_K, _D), jnp.float32)] * 4 + [
            pltpu.VMEM_SHARED((_NP, _D), jnp.float32),
            pltpu.SemaphoreType.DMA,
            pltpu.SemaphoreType.DMA,
            pltpu.SemaphoreType.DMA,
        ],
    )


def _mm1_body(d0_ref, d1_ref, x_ref, w_ref, g_ref, dinv_ref):
    deg = d0_ref[...] + d1_ref[...] + 1.0
    dinv = lax.rsqrt(deg)
    hw = jnp.dot(x_ref[...], w_ref[...], preferred_element_type=jnp.float32)
    g_ref[...] = hw * dinv
    dinv_ref[...] = dinv


def _tc1(d0, d1, x, W1):
    return pl.pallas_call(
        _mm1_body,
        grid=(_N // _R,),
        in_specs=[
            pl.BlockSpec((_R, 1), lambda j: (j, 0)),
            pl.BlockSpec((_R, 1), lambda j: (j, 0)),
            pl.BlockSpec((_R, _D), lambda j: (j, 0)),
            pl.BlockSpec((_D, _D), lambda j: (0, 0)),
        ],
        out_specs=[
            pl.BlockSpec((_R, _D), lambda j: (j, 0)),
            pl.BlockSpec((_R, 1), lambda j: (j, 0)),
        ],
        out_shape=[
            jax.ShapeDtypeStruct((_N, _D), jnp.float32),
            jax.ShapeDtypeStruct((_N, 1), jnp.float32),
        ],
    )(d0, d1, x, W1)


def _mid_body(acc_ref, g_ref, dinv_ref, b_ref, w_ref, out_ref):
    a = acc_ref[...]
    ssum = a[0] + a[1] + g_ref[...]
    h = jnp.maximum(ssum * dinv_ref[...] + b_ref[...], 0.0)
    out_ref[...] = jnp.dot(h, w_ref[...],
                           preferred_element_type=jnp.float32) * dinv_ref[...]


def _tc2(acc, g, dinv, b, W2):
    return pl.pallas_call(
        _mid_body,
        grid=(_N // _R,),
        in_specs=[
            pl.BlockSpec((_NC, _R, _D), lambda j: (0, j, 0)),
            pl.BlockSpec((_R, _D), lambda j: (j, 0)),
            pl.BlockSpec((_R, 1), lambda j: (j, 0)),
            pl.BlockSpec((1, _D), lambda j: (0, 0)),
            pl.BlockSpec((_D, _D), lambda j: (0, 0)),
        ],
        out_specs=pl.BlockSpec((_R, _D), lambda j: (j, 0)),
        out_shape=jax.ShapeDtypeStruct((_N, _D), jnp.float32),
    )(acc, g, dinv, b, W2)


def _fin_body(acc_ref, g_ref, dinv_ref, b_ref, batch_ref, out_ref):
    j = pl.program_id(0)
    a = acc_ref[...]
    h = (a[0] + a[1] + g_ref[...]) * dinv_ref[...] + b_ref[...]
    bb = batch_ref[...]

    @pl.when(j == 0)
    def _():
        out_ref[...] = jnp.full((_G, _D), -jnp.inf, jnp.float32)

    for gi in range(_G):
        vals = jnp.where(bb == gi, h, -jnp.inf)
        m = jnp.max(vals, axis=0)
        out_ref[gi, :] = jnp.maximum(out_ref[gi, :], m)


def _tc3(acc, g, dinv, b, batch2d):
    return pl.pallas_call(
        _fin_body,
        grid=(_N // _R,),
        in_specs=[
            pl.BlockSpec((_NC, _R, _D), lambda j: (0, j, 0)),
            pl.BlockSpec((_R, _D), lambda j: (j, 0)),
            pl.BlockSpec((_R, 1), lambda j: (j, 0)),
            pl.BlockSpec((1, _D), lambda j: (0, 0)),
            pl.BlockSpec((_R, 1), lambda j: (j, 0)),
        ],
        out_specs=pl.BlockSpec((_G, _D), lambda j: (0, 0)),
        out_shape=jax.ShapeDtypeStruct((_G, _D), jnp.float32),
    )(acc, g, dinv, b, batch2d)


def kernel(x, edge_index, batch, W1, b1, W2, b2):
    npad = _EP - _E
    pad_src = (jnp.arange(npad, dtype=jnp.int32) % _N)
    pad_dst = _N + (jnp.arange(npad, dtype=jnp.int32) % (_NP - _N))
    srcf = jnp.concatenate([edge_index[0], pad_src])
    dstf = jnp.concatenate([edge_index[1], pad_dst])
    srcp = srcf.reshape(_NW, _CHT, _K)
    dstp = dstf.reshape(_NW, _CHT, _K)
    eidx = jnp.stack([srcp, dstp], axis=2).reshape(_NW, 2 * _CHT, _K)
    didx = dstf.reshape(_NW, _CHD, _KD)

    zeros_nd = jnp.zeros((_NP, _D), jnp.float32)
    zeros_n1 = jnp.zeros((_NP,), jnp.float32)
    ones_k1 = jnp.ones((_KD,), jnp.float32)

    degp = _deg_call()(didx, ones_k1, zeros_n1)
    d0 = degp[0].reshape(_NP, 1)
    d1 = degp[1].reshape(_NP, 1)
    g1, dinv = _tc1(d0, d1, x, W1)
    acc1 = _edge_call()(g1, eidx, zeros_nd)
    g2 = _tc2(acc1, g1, dinv, b1.reshape(1, _D), W2)
    acc2 = _edge_call()(g2, eidx, zeros_nd)
    out = _tc3(acc2, g2, dinv, b2.reshape(1, _D), batch.reshape(_N, 1))
    return out
